# Initial kernel scaffold; baseline (speedup 1.0000x reference)
import jax
import jax.numpy as jnp
from jax.experimental import pallas as pl


def kernel(x, edge_index, batch, W1a, b1a, W1b, b1b, g1, be1, rm1, rv1, W2a, b2a, W2b, b2b, g2, be2, rm2, rv2, Wf1, bf1, Wf2, bf2):
    raise NotImplementedError("write your pallas kernel here")


# speedup vs baseline: 7.7998x; 7.7998x over previous
import functools

import jax
import jax.numpy as jnp
from jax import lax
from jax.experimental import pallas as pl
from jax.experimental.pallas import tpu as pltpu
from jax.experimental.pallas import tpu_sc as plsc

N = 10000
E = 320000
DIN = 128
DIM = 64
DOUT = 10
G = 64

NC = 2
NS = 16
NW = NC * NS
PW = E // NW
C = 80
KC = PW // C
N_PAD = 10240
ROWS_PER_TILE = N_PAD // NS


@functools.lru_cache(maxsize=1)
def _make_seg_sum():
    mesh = plsc.VectorSubcoreMesh(core_axis_name="c", subcore_axis_name="s",
                                  num_cores=NC, num_subcores=NS)

    @functools.partial(
        pl.kernel,
        mesh=mesh,
        out_type=jax.ShapeDtypeStruct((NC, N_PAD, DIM), jnp.float32),
        scratch_types=[
            pltpu.VMEM((KC, C), jnp.int32),
            pltpu.VMEM((KC, C), jnp.int32),
            pltpu.VMEM((C, DIM), jnp.float32),
            pltpu.VMEM_SHARED((N_PAD, DIM), jnp.float32),
            pltpu.SemaphoreType.DMA,
        ],
        compiler_params=pltpu.CompilerParams(use_tc_tiling_on_sc=False),
    )
    def seg_kernel(y_hbm, src_hbm, dst_hbm, zero_hbm, out_hbm,
                   src_v, dst_v, rows_v, acc, sem):
        c = lax.axis_index("c")
        s = lax.axis_index("s")
        w = s * NC + c
        r0 = s * ROWS_PER_TILE
        pltpu.sync_copy(zero_hbm.at[pl.ds(r0, ROWS_PER_TILE)],
                        acc.at[pl.ds(r0, ROWS_PER_TILE)])
        pltpu.sync_copy(src_hbm.at[w], src_v)
        pltpu.sync_copy(dst_hbm.at[w], dst_v)
        plsc.subcore_barrier()

        def body(j, carry):
            pltpu.async_copy(y_hbm.at[src_v.at[j]], rows_v, sem).wait()
            pltpu.sync_copy(rows_v, acc.at[dst_v.at[j]], add=True)
            return carry

        lax.fori_loop(0, KC, body, 0)
        plsc.subcore_barrier()
        pltpu.sync_copy(acc.at[pl.ds(r0, ROWS_PER_TILE)],
                        out_hbm.at[c, pl.ds(r0, ROWS_PER_TILE)])

    return seg_kernel


BN = 1000
NB = N // BN


def _mm_body(x_ref, w_ref, o_ref):
    o_ref[:] = jnp.dot(x_ref[:], w_ref[:], preferred_element_type=jnp.float32)


def _tc_in_matmul(x, W1a):
    return pl.pallas_call(
        _mm_body,
        grid=(NB,),
        in_specs=[
            pl.BlockSpec((BN, DIN), lambda i: (i, 0)),
            pl.BlockSpec((DIN, DIM), lambda i: (0, 0)),
        ],
        out_specs=pl.BlockSpec((BN, DIM), lambda i: (i, 0)),
        out_shape=jax.ShapeDtypeStruct((N, DIM), jnp.float32),
    )(x, W1a)


def _stage2_body(p0_ref, p1_ref, y_ref, b1a_ref, w1b_ref, b1b_ref,
                 sc1_ref, sh1_ref, w2a_ref, h_ref, z_ref):
    pre = p0_ref[:] + p1_ref[:] + y_ref[:] + b1a_ref[:]
    a = jnp.maximum(pre, 0.0)
    t = jnp.dot(a, w1b_ref[:], preferred_element_type=jnp.float32) + b1b_ref[:]
    t = jnp.maximum(t, 0.0)
    h = t * sc1_ref[:] + sh1_ref[:]
    h_ref[:] = h
    z_ref[:] = jnp.dot(h, w2a_ref[:], preferred_element_type=jnp.float32)


def _tc_stage2(p0, p1, y1, b1a, W1b, b1b, sc1, sh1, W2a):
    vec = lambda: pl.BlockSpec((1, DIM), lambda i: (0, 0))
    mat = lambda: pl.BlockSpec((DIM, DIM), lambda i: (0, 0))
    blk = lambda: pl.BlockSpec((BN, DIM), lambda i: (i, 0))
    return pl.pallas_call(
        _stage2_body,
        grid=(NB,),
        in_specs=[blk(), blk(), blk(), vec(), mat(), vec(), vec(), vec(), mat()],
        out_specs=[blk(), blk()],
        out_shape=[
            jax.ShapeDtypeStruct((N, DIM), jnp.float32),
            jax.ShapeDtypeStruct((N, DIM), jnp.float32),
        ],
    )(p0, p1, y1, b1a, W1b, b1b, sc1, sh1, W2a)


def _stage3_body(p0_ref, p1_ref, z_ref, b2a_ref, w2b_ref, b2b_ref,
                 sc2_ref, sh2_ref, batch_ref, wf1_ref, bf1_ref,
                 wf2_ref, bf2_ref, out_ref, sums_ref, cnt_ref):
    i = pl.program_id(0)

    @pl.when(i == 0)
    def _():
        sums_ref[:] = jnp.zeros_like(sums_ref)
        cnt_ref[:] = jnp.zeros_like(cnt_ref)

    pre = p0_ref[:] + p1_ref[:] + z_ref[:] + b2a_ref[:]
    a = jnp.maximum(pre, 0.0)
    t = jnp.dot(a, w2b_ref[:], preferred_element_type=jnp.float32) + b2b_ref[:]
    t = jnp.maximum(t, 0.0)
    h2 = t * sc2_ref[:] + sh2_ref[:]

    b = batch_ref[0, 0, :]
    onehot = (b[:, None] == lax.broadcasted_iota(jnp.int32, (BN, G), 1))
    onehot = onehot.astype(jnp.float32)
    sums_ref[:] += lax.dot_general(
        onehot, h2, (((0,), (0,)), ((), ())),
        preferred_element_type=jnp.float32)
    cnt_ref[:] += lax.dot_general(
        onehot, jnp.ones_like(h2), (((0,), (0,)), ((), ())),
        preferred_element_type=jnp.float32)

    @pl.when(i == NB - 1)
    def _():
        pooled = sums_ref[:] / jnp.maximum(cnt_ref[:], 1.0)
        h3 = jnp.maximum(
            jnp.dot(pooled, wf1_ref[:], preferred_element_type=jnp.float32)
            + bf1_ref[:], 0.0)
        out_ref[:] = (jnp.dot(h3, wf2_ref[:],
                              preferred_element_type=jnp.float32)
                      + bf2_ref[:])


def _tc_stage3(p0, p1, z, b2a, W2b, b2b, sc2, sh2, batch3d,
               Wf1, bf1, Wf2, bf2):
    vec = lambda: pl.BlockSpec((1, DIM), lambda i: (0, 0))
    mat = lambda: pl.BlockSpec((DIM, DIM), lambda i: (0, 0))
    blk = lambda: pl.BlockSpec((BN, DIM), lambda i: (i, 0))
    return pl.pallas_call(
        _stage3_body,
        grid=(NB,),
        in_specs=[
            blk(), blk(), blk(), vec(), mat(), vec(), vec(), vec(),
            pl.BlockSpec((1, 1, BN), lambda i: (i, 0, 0)),
            mat(), vec(),
            pl.BlockSpec((DIM, DOUT), lambda i: (0, 0)),
            pl.BlockSpec((1, DOUT), lambda i: (0, 0)),
        ],
        out_specs=pl.BlockSpec((G, DOUT), lambda i: (0, 0)),
        out_shape=jax.ShapeDtypeStruct((G, DOUT), jnp.float32),
        scratch_shapes=[
            pltpu.VMEM((G, DIM), jnp.float32),
            pltpu.VMEM((G, DIM), jnp.float32),
        ],
    )(p0, p1, z, b2a, W2b, b2b, sc2, sh2, batch3d, Wf1, bf1, Wf2, bf2)


def kernel(x, edge_index, batch, W1a, b1a, W1b, b1b, g1, be1, rm1, rv1,
           W2a, b2a, W2b, b2b, g2, be2, rm2, rv2, Wf1, bf1, Wf2, bf2):
    src3d = edge_index[0].reshape(NW, KC, C)
    dst3d = edge_index[1].reshape(NW, KC, C)
    zeros = jnp.zeros((N_PAD, DIM), jnp.float32)
    batch3d = batch.reshape(NB, 1, BN)

    sc1 = (g1 / jnp.sqrt(rv1 + 1e-5)).reshape(1, DIM)
    sh1 = (be1 - rm1 * g1 / jnp.sqrt(rv1 + 1e-5)).reshape(1, DIM)
    sc2 = (g2 / jnp.sqrt(rv2 + 1e-5)).reshape(1, DIM)
    sh2 = (be2 - rm2 * g2 / jnp.sqrt(rv2 + 1e-5)).reshape(1, DIM)

    seg_sum = _make_seg_sum()
    y1 = _tc_in_matmul(x, W1a)
    p = seg_sum(y1, src3d, dst3d, zeros)
    h, z = _tc_stage2(p[0, :N], p[1, :N], y1, b1a.reshape(1, DIM), W1b,
                      b1b.reshape(1, DIM), sc1, sh1, W2a)
    p2 = seg_sum(z, src3d, dst3d, zeros)
    out = _tc_stage3(p2[0, :N], p2[1, :N], z, b2a.reshape(1, DIM), W2b,
                     b2b.reshape(1, DIM), sc2, sh2, batch3d,
                     Wf1, bf1.reshape(1, DIM), Wf2, bf2.reshape(1, DOUT))
    return out


# speedup vs baseline: 13.0421x; 1.6721x over previous
import functools

import jax
import jax.numpy as jnp
from jax import lax
from jax.experimental import pallas as pl
from jax.experimental.pallas import tpu as pltpu
from jax.experimental.pallas import tpu_sc as plsc

N = 10000
E = 320000
DIN = 128
DIM = 64
DOUT = 10
G = 64

NC = 2
NS = 16
NW = NC * NS
PW = E // NW
C = 500
KC = PW // C
N_PAD = 10240
ROWS_PER_TILE = N_PAD // NS


@functools.lru_cache(maxsize=1)
def _make_seg_sum():
    mesh = plsc.VectorSubcoreMesh(core_axis_name="c", subcore_axis_name="s",
                                  num_cores=NC, num_subcores=NS)

    @functools.partial(
        pl.kernel,
        mesh=mesh,
        out_type=jax.ShapeDtypeStruct((NC, N_PAD, DIM), jnp.float32),
        scratch_types=[
            pltpu.VMEM((KC, C), jnp.int32),
            pltpu.VMEM((KC, C), jnp.int32),
            pltpu.VMEM((C, DIM), jnp.float32),
            pltpu.VMEM((C, DIM), jnp.float32),
            pltpu.VMEM_SHARED((N_PAD, DIM), jnp.float32),
            pltpu.SemaphoreType.DMA,
            pltpu.SemaphoreType.DMA,
            pltpu.SemaphoreType.DMA,
            pltpu.SemaphoreType.DMA,
        ],
        compiler_params=pltpu.CompilerParams(use_tc_tiling_on_sc=False),
    )
    def seg_kernel(y_hbm, src_hbm, dst_hbm, zero_hbm, out_hbm,
                   src_v, dst_v, rows0, rows1, acc, g0, g1, s0, s1):
        c = lax.axis_index("c")
        s = lax.axis_index("s")
        w = s * NC + c
        r0 = s * ROWS_PER_TILE
        pltpu.sync_copy(zero_hbm.at[pl.ds(r0, ROWS_PER_TILE)],
                        acc.at[pl.ds(r0, ROWS_PER_TILE)])
        pltpu.sync_copy(src_hbm.at[w], src_v)
        pltpu.sync_copy(dst_hbm.at[w], dst_v)
        plsc.subcore_barrier()

        def gather(j, buf, sem):
            pltpu.async_copy(y_hbm.at[src_v.at[j]], buf, sem)

        def wait_gather(j, buf, sem):
            pltpu.make_async_copy(y_hbm.at[src_v.at[j]], buf, sem).wait()

        def scatter(j, buf, sem):
            pltpu.async_copy(buf, acc.at[dst_v.at[j]], sem, add=True)

        def wait_scatter(j, buf, sem):
            pltpu.make_async_copy(buf, acc.at[dst_v.at[j]], sem).wait()

        gather(0, rows0, g0)

        def body(i, carry):
            j = 2 * i
            wait_gather(j, rows0, g0)
            scatter(j, rows0, s0)

            @pl.when(i > 0)
            def _():
                wait_scatter(j - 1, rows1, s1)

            gather(j + 1, rows1, g1)
            wait_gather(j + 1, rows1, g1)
            scatter(j + 1, rows1, s1)
            wait_scatter(j, rows0, s0)

            @pl.when(j + 2 < KC)
            def _():
                gather(j + 2, rows0, g0)
            return carry

        lax.fori_loop(0, KC // 2, body, 0)
        wait_scatter(KC - 1, rows1, s1)
        plsc.subcore_barrier()
        pltpu.sync_copy(acc.at[pl.ds(r0, ROWS_PER_TILE)],
                        out_hbm.at[c, pl.ds(r0, ROWS_PER_TILE)])

    return seg_kernel


BN = 1000
NB = N // BN


def _mm_body(x_ref, w_ref, o_ref):
    o_ref[:] = jnp.dot(x_ref[:], w_ref[:], preferred_element_type=jnp.float32)


def _tc_in_matmul(x, W1a):
    return pl.pallas_call(
        _mm_body,
        grid=(NB,),
        in_specs=[
            pl.BlockSpec((BN, DIN), lambda i: (i, 0)),
            pl.BlockSpec((DIN, DIM), lambda i: (0, 0)),
        ],
        out_specs=pl.BlockSpec((BN, DIM), lambda i: (i, 0)),
        out_shape=jax.ShapeDtypeStruct((N, DIM), jnp.float32),
    )(x, W1a)


def _stage2_body(p0_ref, p1_ref, y_ref, b1a_ref, w1b_ref, b1b_ref,
                 sc1_ref, sh1_ref, w2a_ref, h_ref, z_ref):
    pre = p0_ref[:] + p1_ref[:] + y_ref[:] + b1a_ref[:]
    a = jnp.maximum(pre, 0.0)
    t = jnp.dot(a, w1b_ref[:], preferred_element_type=jnp.float32) + b1b_ref[:]
    t = jnp.maximum(t, 0.0)
    h = t * sc1_ref[:] + sh1_ref[:]
    h_ref[:] = h
    z_ref[:] = jnp.dot(h, w2a_ref[:], preferred_element_type=jnp.float32)


def _tc_stage2(p0, p1, y1, b1a, W1b, b1b, sc1, sh1, W2a):
    vec = lambda: pl.BlockSpec((1, DIM), lambda i: (0, 0))
    mat = lambda: pl.BlockSpec((DIM, DIM), lambda i: (0, 0))
    blk = lambda: pl.BlockSpec((BN, DIM), lambda i: (i, 0))
    return pl.pallas_call(
        _stage2_body,
        grid=(NB,),
        in_specs=[blk(), blk(), blk(), vec(), mat(), vec(), vec(), vec(), mat()],
        out_specs=[blk(), blk()],
        out_shape=[
            jax.ShapeDtypeStruct((N, DIM), jnp.float32),
            jax.ShapeDtypeStruct((N, DIM), jnp.float32),
        ],
    )(p0, p1, y1, b1a, W1b, b1b, sc1, sh1, W2a)


def _stage3_body(p0_ref, p1_ref, z_ref, b2a_ref, w2b_ref, b2b_ref,
                 sc2_ref, sh2_ref, batch_ref, wf1_ref, bf1_ref,
                 wf2_ref, bf2_ref, out_ref, sums_ref, cnt_ref):
    i = pl.program_id(0)

    @pl.when(i == 0)
    def _():
        sums_ref[:] = jnp.zeros_like(sums_ref)
        cnt_ref[:] = jnp.zeros_like(cnt_ref)

    pre = p0_ref[:] + p1_ref[:] + z_ref[:] + b2a_ref[:]
    a = jnp.maximum(pre, 0.0)
    t = jnp.dot(a, w2b_ref[:], preferred_element_type=jnp.float32) + b2b_ref[:]
    t = jnp.maximum(t, 0.0)
    h2 = t * sc2_ref[:] + sh2_ref[:]

    b = batch_ref[0, 0, :]
    onehot = (b[:, None] == lax.broadcasted_iota(jnp.int32, (BN, G), 1))
    onehot = onehot.astype(jnp.float32)
    sums_ref[:] += lax.dot_general(
        onehot, h2, (((0,), (0,)), ((), ())),
        preferred_element_type=jnp.float32)
    cnt_ref[:] += lax.dot_general(
        onehot, jnp.ones_like(h2), (((0,), (0,)), ((), ())),
        preferred_element_type=jnp.float32)

    @pl.when(i == NB - 1)
    def _():
        pooled = sums_ref[:] / jnp.maximum(cnt_ref[:], 1.0)
        h3 = jnp.maximum(
            jnp.dot(pooled, wf1_ref[:], preferred_element_type=jnp.float32)
            + bf1_ref[:], 0.0)
        out_ref[:] = (jnp.dot(h3, wf2_ref[:],
                              preferred_element_type=jnp.float32)
                      + bf2_ref[:])


def _tc_stage3(p0, p1, z, b2a, W2b, b2b, sc2, sh2, batch3d,
               Wf1, bf1, Wf2, bf2):
    vec = lambda: pl.BlockSpec((1, DIM), lambda i: (0, 0))
    mat = lambda: pl.BlockSpec((DIM, DIM), lambda i: (0, 0))
    blk = lambda: pl.BlockSpec((BN, DIM), lambda i: (i, 0))
    return pl.pallas_call(
        _stage3_body,
        grid=(NB,),
        in_specs=[
            blk(), blk(), blk(), vec(), mat(), vec(), vec(), vec(),
            pl.BlockSpec((1, 1, BN), lambda i: (i, 0, 0)),
            mat(), vec(),
            pl.BlockSpec((DIM, DOUT), lambda i: (0, 0)),
            pl.BlockSpec((1, DOUT), lambda i: (0, 0)),
        ],
        out_specs=pl.BlockSpec((G, DOUT), lambda i: (0, 0)),
        out_shape=jax.ShapeDtypeStruct((G, DOUT), jnp.float32),
        scratch_shapes=[
            pltpu.VMEM((G, DIM), jnp.float32),
            pltpu.VMEM((G, DIM), jnp.float32),
        ],
    )(p0, p1, z, b2a, W2b, b2b, sc2, sh2, batch3d, Wf1, bf1, Wf2, bf2)


def kernel(x, edge_index, batch, W1a, b1a, W1b, b1b, g1, be1, rm1, rv1,
           W2a, b2a, W2b, b2b, g2, be2, rm2, rv2, Wf1, bf1, Wf2, bf2):
    src3d = edge_index[0].reshape(NW, KC, C)
    dst3d = edge_index[1].reshape(NW, KC, C)
    zeros = jnp.zeros((N_PAD, DIM), jnp.float32)
    batch3d = batch.reshape(NB, 1, BN)

    sc1 = (g1 / jnp.sqrt(rv1 + 1e-5)).reshape(1, DIM)
    sh1 = (be1 - rm1 * g1 / jnp.sqrt(rv1 + 1e-5)).reshape(1, DIM)
    sc2 = (g2 / jnp.sqrt(rv2 + 1e-5)).reshape(1, DIM)
    sh2 = (be2 - rm2 * g2 / jnp.sqrt(rv2 + 1e-5)).reshape(1, DIM)

    seg_sum = _make_seg_sum()
    y1 = _tc_in_matmul(x, W1a)
    p = seg_sum(y1, src3d, dst3d, zeros)
    h, z = _tc_stage2(p[0, :N], p[1, :N], y1, b1a.reshape(1, DIM), W1b,
                      b1b.reshape(1, DIM), sc1, sh1, W2a)
    p2 = seg_sum(z, src3d, dst3d, zeros)
    out = _tc_stage3(p2[0, :N], p2[1, :N], z, b2a.reshape(1, DIM), W2b,
                     b2b.reshape(1, DIM), sc2, sh2, batch3d,
                     Wf1, bf1.reshape(1, DIM), Wf2, bf2.reshape(1, DOUT))
    return out


# speedup vs baseline: 15.5248x; 1.1904x over previous
import functools

import jax
import jax.numpy as jnp
from jax import lax
from jax.experimental import pallas as pl
from jax.experimental.pallas import tpu as pltpu
from jax.experimental.pallas import tpu_sc as plsc

N = 10000
E = 320000
DIN = 128
DIM = 64
DOUT = 10
G = 64

NC = 2
NS = 16
NW = NC * NS
PW = E // NW
C = 250
KC = PW // C
NBUF = 4
N_PAD = 10240
ROWS_PER_TILE = N_PAD // NS


@functools.lru_cache(maxsize=1)
def _make_seg_sum():
    mesh = plsc.VectorSubcoreMesh(core_axis_name="c", subcore_axis_name="s",
                                  num_cores=NC, num_subcores=NS)

    @functools.partial(
        pl.kernel,
        mesh=mesh,
        out_type=jax.ShapeDtypeStruct((NC, N_PAD, DIM), jnp.float32),
        scratch_types=[
            pltpu.VMEM((KC, C), jnp.int32),
            pltpu.VMEM((KC, C), jnp.int32),
        ]
        + [pltpu.VMEM((C, DIM), jnp.float32) for _ in range(NBUF)]
        + [pltpu.VMEM_SHARED((N_PAD, DIM), jnp.float32)]
        + [pltpu.SemaphoreType.DMA for _ in range(2 * NBUF)],
        compiler_params=pltpu.CompilerParams(use_tc_tiling_on_sc=False),
    )
    def seg_kernel(y_hbm, ei_hbm, zero_hbm, out_hbm, src_v, dst_v, *rest):
        rows = rest[:NBUF]
        acc = rest[NBUF]
        gsem = rest[NBUF + 1:NBUF + 1 + NBUF]
        ssem = rest[NBUF + 1 + NBUF:]
        c = lax.axis_index("c")
        s = lax.axis_index("s")
        w = s * NC + c
        r0 = s * ROWS_PER_TILE
        pltpu.sync_copy(zero_hbm.at[pl.ds(r0, ROWS_PER_TILE)],
                        acc.at[pl.ds(r0, ROWS_PER_TILE)])
        pltpu.sync_copy(ei_hbm.at[0, w], src_v)
        pltpu.sync_copy(ei_hbm.at[1, w], dst_v)
        plsc.subcore_barrier()

        def gather(j, b):
            pltpu.async_copy(y_hbm.at[src_v.at[j]], rows[b], gsem[b])

        def wait_gather(j, b):
            pltpu.make_async_copy(y_hbm.at[src_v.at[j]], rows[b],
                                  gsem[b]).wait()

        def scatter(j, b):
            pltpu.async_copy(rows[b], acc.at[dst_v.at[j]], ssem[b], add=True)

        def wait_scatter(j, b):
            pltpu.make_async_copy(rows[b], acc.at[dst_v.at[j]],
                                  ssem[b]).wait()

        gather(0, 0)
        gather(1, 1)

        def body(i, carry):
            j0 = NBUF * i
            for b in range(NBUF):
                j = j0 + b
                wait_gather(j, b)
                scatter(j, b)
                jp = j - (NBUF - 2)
                bp = (b - 2) % NBUF
                bn = (b + 2) % NBUF

                @pl.when(jp >= 0)
                def _():
                    wait_scatter(jp, bp)

                @pl.when(j + 2 < KC)
                def _():
                    gather(j + 2, bn)
            return carry

        lax.fori_loop(0, KC // NBUF, body, 0)
        wait_scatter(KC - 2, (KC - 2) % NBUF)
        wait_scatter(KC - 1, (KC - 1) % NBUF)
        plsc.subcore_barrier()
        pltpu.sync_copy(acc.at[pl.ds(r0, ROWS_PER_TILE)],
                        out_hbm.at[c, pl.ds(r0, ROWS_PER_TILE)])

    return seg_kernel


BN = 2000
NB = N // BN


def _mm_body(x_ref, w_ref, o_ref):
    o_ref[:] = jnp.dot(x_ref[:], w_ref[:], preferred_element_type=jnp.float32)


def _tc_in_matmul(x, W1a):
    return pl.pallas_call(
        _mm_body,
        grid=(NB,),
        in_specs=[
            pl.BlockSpec((BN, DIN), lambda i: (i, 0)),
            pl.BlockSpec((DIN, DIM), lambda i: (0, 0)),
        ],
        out_specs=pl.BlockSpec((BN, DIM), lambda i: (i, 0)),
        out_shape=jax.ShapeDtypeStruct((N, DIM), jnp.float32),
    )(x, W1a)


def _stage2_body(p0_ref, p1_ref, y_ref, b1a_ref, w1b_ref, b1b_ref,
                 sc1_ref, sh1_ref, w2a_ref, h_ref, z_ref):
    pre = p0_ref[0] + p1_ref[0] + y_ref[:] + b1a_ref[:]
    a = jnp.maximum(pre, 0.0)
    t = jnp.dot(a, w1b_ref[:], preferred_element_type=jnp.float32) + b1b_ref[:]
    t = jnp.maximum(t, 0.0)
    h = t * sc1_ref[:] + sh1_ref[:]
    h_ref[:] = h
    z_ref[:] = jnp.dot(h, w2a_ref[:], preferred_element_type=jnp.float32)


def _tc_stage2(p, y1, b1a, W1b, b1b, sc1, sh1, W2a):
    vec = lambda: pl.BlockSpec((1, DIM), lambda i: (0, 0))
    mat = lambda: pl.BlockSpec((DIM, DIM), lambda i: (0, 0))
    blk = lambda: pl.BlockSpec((BN, DIM), lambda i: (i, 0))
    p0s = pl.BlockSpec((1, BN, DIM), lambda i: (0, i, 0))
    p1s = pl.BlockSpec((1, BN, DIM), lambda i: (1, i, 0))
    return pl.pallas_call(
        _stage2_body,
        grid=(NB,),
        in_specs=[p0s, p1s, blk(), vec(), mat(), vec(), vec(), vec(), mat()],
        out_specs=[blk(), blk()],
        out_shape=[
            jax.ShapeDtypeStruct((N, DIM), jnp.float32),
            jax.ShapeDtypeStruct((N, DIM), jnp.float32),
        ],
    )(p, p, y1, b1a, W1b, b1b, sc1, sh1, W2a)


def _stage3_body(p0_ref, p1_ref, z_ref, b2a_ref, w2b_ref, b2b_ref,
                 sc2_ref, sh2_ref, batch_ref, wf1_ref, bf1_ref,
                 wf2_ref, bf2_ref, out_ref, sums_ref, cnt_ref):
    i = pl.program_id(0)

    @pl.when(i == 0)
    def _():
        sums_ref[:] = jnp.zeros_like(sums_ref)
        cnt_ref[:] = jnp.zeros_like(cnt_ref)

    pre = p0_ref[0] + p1_ref[0] + z_ref[:] + b2a_ref[:]
    a = jnp.maximum(pre, 0.0)
    t = jnp.dot(a, w2b_ref[:], preferred_element_type=jnp.float32) + b2b_ref[:]
    t = jnp.maximum(t, 0.0)
    h2 = t * sc2_ref[:] + sh2_ref[:]

    b = batch_ref[0, 0, :]
    onehot = (b[:, None] == lax.broadcasted_iota(jnp.int32, (BN, G), 1))
    onehot = onehot.astype(jnp.float32)
    sums_ref[:] += lax.dot_general(
        onehot, h2, (((0,), (0,)), ((), ())),
        preferred_element_type=jnp.float32)
    cnt_ref[:] += lax.dot_general(
        onehot, jnp.ones_like(h2), (((0,), (0,)), ((), ())),
        preferred_element_type=jnp.float32)

    @pl.when(i == NB - 1)
    def _():
        pooled = sums_ref[:] / jnp.maximum(cnt_ref[:], 1.0)
        h3 = jnp.maximum(
            jnp.dot(pooled, wf1_ref[:], preferred_element_type=jnp.float32)
            + bf1_ref[:], 0.0)
        out_ref[:] = (jnp.dot(h3, wf2_ref[:],
                              preferred_element_type=jnp.float32)
                      + bf2_ref[:])


def _tc_stage3(p, z, b2a, W2b, b2b, sc2, sh2, batch3d,
               Wf1, bf1, Wf2, bf2):
    vec = lambda: pl.BlockSpec((1, DIM), lambda i: (0, 0))
    mat = lambda: pl.BlockSpec((DIM, DIM), lambda i: (0, 0))
    blk = lambda: pl.BlockSpec((BN, DIM), lambda i: (i, 0))
    p0s = pl.BlockSpec((1, BN, DIM), lambda i: (0, i, 0))
    p1s = pl.BlockSpec((1, BN, DIM), lambda i: (1, i, 0))
    return pl.pallas_call(
        _stage3_body,
        grid=(NB,),
        in_specs=[
            p0s, p1s, blk(), vec(), mat(), vec(), vec(), vec(),
            pl.BlockSpec((1, 1, BN), lambda i: (i, 0, 0)),
            mat(), vec(),
            pl.BlockSpec((DIM, DOUT), lambda i: (0, 0)),
            pl.BlockSpec((1, DOUT), lambda i: (0, 0)),
        ],
        out_specs=pl.BlockSpec((G, DOUT), lambda i: (0, 0)),
        out_shape=jax.ShapeDtypeStruct((G, DOUT), jnp.float32),
        scratch_shapes=[
            pltpu.VMEM((G, DIM), jnp.float32),
            pltpu.VMEM((G, DIM), jnp.float32),
        ],
    )(p, p, z, b2a, W2b, b2b, sc2, sh2, batch3d, Wf1, bf1, Wf2, bf2)


def kernel(x, edge_index, batch, W1a, b1a, W1b, b1b, g1, be1, rm1, rv1,
           W2a, b2a, W2b, b2b, g2, be2, rm2, rv2, Wf1, bf1, Wf2, bf2):
    ei4d = edge_index.reshape(2, NW, KC, C)
    zeros = jnp.zeros((N_PAD, DIM), jnp.float32)
    batch3d = batch.reshape(NB, 1, BN)

    sc1 = (g1 / jnp.sqrt(rv1 + 1e-5)).reshape(1, DIM)
    sh1 = (be1 - rm1 * g1 / jnp.sqrt(rv1 + 1e-5)).reshape(1, DIM)
    sc2 = (g2 / jnp.sqrt(rv2 + 1e-5)).reshape(1, DIM)
    sh2 = (be2 - rm2 * g2 / jnp.sqrt(rv2 + 1e-5)).reshape(1, DIM)

    seg_sum = _make_seg_sum()
    y1 = _tc_in_matmul(x, W1a)
    p = seg_sum(y1, ei4d, zeros)
    h, z = _tc_stage2(p, y1, b1a.reshape(1, DIM), W1b,
                      b1b.reshape(1, DIM), sc1, sh1, W2a)
    p2 = seg_sum(z, ei4d, zeros)
    out = _tc_stage3(p2, z, b2a.reshape(1, DIM), W2b,
                     b2b.reshape(1, DIM), sc2, sh2, batch3d,
                     Wf1, bf1.reshape(1, DIM), Wf2, bf2.reshape(1, DOUT))
    return out


# speedup vs baseline: 18.5081x; 1.1922x over previous
import functools

import jax
import jax.numpy as jnp
from jax import lax
from jax.experimental import pallas as pl
from jax.experimental.pallas import tpu as pltpu
from jax.experimental.pallas import tpu_sc as plsc

N = 10000
E = 320000
DIN = 128
DIM = 64
DOUT = 10
G = 64

NC = 2
NS = 16
NW = NC * NS
PW = E // NW
C = 250
KC = PW // C
NBUF = 4
N_PAD = 10240
ROWS_PER_TILE = N_PAD // NS


@functools.lru_cache(maxsize=1)
def _make_seg_sum():
    mesh = plsc.VectorSubcoreMesh(core_axis_name="c", subcore_axis_name="s",
                                  num_cores=NC, num_subcores=NS)

    @functools.partial(
        pl.kernel,
        mesh=mesh,
        out_type=jax.ShapeDtypeStruct((NC, N_PAD, DIM), jnp.float32),
        scratch_types=[
            pltpu.VMEM((KC, C), jnp.int32),
            pltpu.VMEM((KC, C), jnp.int32),
        ]
        + [pltpu.VMEM((C, DIM), jnp.float32) for _ in range(NBUF)]
        + [pltpu.VMEM_SHARED((N_PAD, DIM), jnp.float32)]
        + [pltpu.SemaphoreType.DMA for _ in range(2 * NBUF)],
        compiler_params=pltpu.CompilerParams(use_tc_tiling_on_sc=False),
    )
    def seg_kernel(y_hbm, ei_hbm, zero_hbm, out_hbm, src_v, dst_v, *rest):
        rows = rest[:NBUF]
        acc = rest[NBUF]
        gsem = rest[NBUF + 1:NBUF + 1 + NBUF]
        ssem = rest[NBUF + 1 + NBUF:]
        c = lax.axis_index("c")
        s = lax.axis_index("s")
        w = s * NC + c
        r0 = s * ROWS_PER_TILE
        pltpu.sync_copy(zero_hbm.at[pl.ds(r0, ROWS_PER_TILE)],
                        acc.at[pl.ds(r0, ROWS_PER_TILE)])
        pltpu.sync_copy(ei_hbm.at[0, w], src_v)
        pltpu.sync_copy(ei_hbm.at[1, w], dst_v)
        plsc.subcore_barrier()

        def gather(j, b):
            pltpu.async_copy(y_hbm.at[src_v.at[j]], rows[b], gsem[b])

        def wait_gather(j, b):
            pltpu.make_async_copy(y_hbm.at[src_v.at[j]], rows[b],
                                  gsem[b]).wait()

        def scatter(j, b):
            pltpu.async_copy(rows[b], acc.at[dst_v.at[j]], ssem[b], add=True)

        def wait_scatter(j, b):
            pltpu.make_async_copy(rows[b], acc.at[dst_v.at[j]],
                                  ssem[b]).wait()

        gather(0, 0)
        gather(1, 1)

        def body(i, carry):
            j0 = NBUF * i
            for b in range(NBUF):
                j = j0 + b
                wait_gather(j, b)
                scatter(j, b)
                jp = j - (NBUF - 2)
                bp = (b - 2) % NBUF
                bn = (b + 2) % NBUF

                @pl.when(jp >= 0)
                def _():
                    wait_scatter(jp, bp)

                @pl.when(j + 2 < KC)
                def _():
                    gather(j + 2, bn)
            return carry

        lax.fori_loop(0, KC // NBUF, body, 0)
        wait_scatter(KC - 2, (KC - 2) % NBUF)
        wait_scatter(KC - 1, (KC - 1) % NBUF)
        plsc.subcore_barrier()
        pltpu.sync_copy(acc.at[pl.ds(r0, ROWS_PER_TILE)],
                        out_hbm.at[c, pl.ds(r0, ROWS_PER_TILE)])

    return seg_kernel


H = N // 2
D2 = 2 * DIM
BN = 1000
NB = H // BN
NPH = N_PAD // 2


def _mm_body(xlo_ref, xhi_ref, w_ref, o_ref):
    ylo = jnp.dot(xlo_ref[:], w_ref[:], preferred_element_type=jnp.float32)
    yhi = jnp.dot(xhi_ref[:], w_ref[:], preferred_element_type=jnp.float32)
    o_ref[:] = jnp.concatenate([ylo, yhi], axis=-1)


def _tc_in_matmul(x, W1a):
    return pl.pallas_call(
        _mm_body,
        grid=(NB,),
        in_specs=[
            pl.BlockSpec((BN, DIN), lambda i: (i, 0)),
            pl.BlockSpec((BN, DIN), lambda i: (i + H // BN, 0)),
            pl.BlockSpec((DIN, DIM), lambda i: (0, 0)),
        ],
        out_specs=pl.BlockSpec((BN, D2), lambda i: (i, 0)),
        out_shape=jax.ShapeDtypeStruct((H, D2), jnp.float32),
    )(x, x, W1a)


def _stage2_body(p0_ref, p1_ref, y_ref, b1a_ref, w1b_ref, b1b_ref,
                 sc1_ref, sh1_ref, w2a_ref, z_ref):
    pre = p0_ref[0] + p1_ref[0] + y_ref[:] + b1a_ref[:]
    a = jnp.maximum(pre, 0.0)
    t = jnp.dot(a, w1b_ref[:], preferred_element_type=jnp.float32) + b1b_ref[:]
    t = jnp.maximum(t, 0.0)
    h = t * sc1_ref[:] + sh1_ref[:]
    z_ref[:] = jnp.dot(h, w2a_ref[:], preferred_element_type=jnp.float32)


def _tc_stage2(pp, y1, b1a2, W1b_bd, b1b2, sc12, sh12, W2a_bd):
    vec = lambda: pl.BlockSpec((1, D2), lambda i: (0, 0))
    mat = lambda: pl.BlockSpec((D2, D2), lambda i: (0, 0))
    blk = lambda: pl.BlockSpec((BN, D2), lambda i: (i, 0))
    p0s = pl.BlockSpec((1, BN, D2), lambda i: (0, i, 0))
    p1s = pl.BlockSpec((1, BN, D2), lambda i: (1, i, 0))
    return pl.pallas_call(
        _stage2_body,
        grid=(NB,),
        in_specs=[p0s, p1s, blk(), vec(), mat(), vec(), vec(), vec(), mat()],
        out_specs=blk(),
        out_shape=jax.ShapeDtypeStruct((H, D2), jnp.float32),
    )(pp, pp, y1, b1a2, W1b_bd, b1b2, sc12, sh12, W2a_bd)


def _stage3_body(p0_ref, p1_ref, z_ref, b2a_ref, w2b_ref, b2b_ref,
                 sc2_ref, sh2_ref, blo_ref, bhi_ref, wf1_ref, bf1_ref,
                 wf2_ref, bf2_ref, out_ref, sums_ref, cnt_ref):
    i = pl.program_id(0)

    @pl.when(i == 0)
    def _():
        sums_ref[:] = jnp.zeros_like(sums_ref)
        cnt_ref[:] = jnp.zeros_like(cnt_ref)

    pre = p0_ref[0] + p1_ref[0] + z_ref[:] + b2a_ref[:]
    a = jnp.maximum(pre, 0.0)
    t = jnp.dot(a, w2b_ref[:], preferred_element_type=jnp.float32) + b2b_ref[:]
    t = jnp.maximum(t, 0.0)
    h2 = t * sc2_ref[:] + sh2_ref[:]
    h2lo = h2[:, :DIM]
    h2hi = h2[:, DIM:]

    iota = lax.broadcasted_iota(jnp.int32, (BN, G), 1)
    ohlo = (blo_ref[0, 0, :][:, None] == iota).astype(jnp.float32)
    ohhi = (bhi_ref[0, 0, :][:, None] == iota).astype(jnp.float32)
    cdim = (((0,), (0,)), ((), ()))
    sums_ref[:] += (
        lax.dot_general(ohlo, h2lo, cdim, preferred_element_type=jnp.float32)
        + lax.dot_general(ohhi, h2hi, cdim,
                          preferred_element_type=jnp.float32))
    ones = jnp.ones((BN, DIM), jnp.float32)
    cnt_ref[:] += (
        lax.dot_general(ohlo, ones, cdim, preferred_element_type=jnp.float32)
        + lax.dot_general(ohhi, ones, cdim,
                          preferred_element_type=jnp.float32))

    @pl.when(i == NB - 1)
    def _():
        pooled = sums_ref[:] / jnp.maximum(cnt_ref[:], 1.0)
        h3 = jnp.maximum(
            jnp.dot(pooled, wf1_ref[:], preferred_element_type=jnp.float32)
            + bf1_ref[:], 0.0)
        out_ref[:] = (jnp.dot(h3, wf2_ref[:],
                              preferred_element_type=jnp.float32)
                      + bf2_ref[:])


def _tc_stage3(pp, z, b2a2, W2b_bd, b2b2, sc22, sh22, batch3d,
               Wf1, bf1, Wf2, bf2):
    vec = lambda: pl.BlockSpec((1, D2), lambda i: (0, 0))
    mat = lambda: pl.BlockSpec((D2, D2), lambda i: (0, 0))
    blk = lambda: pl.BlockSpec((BN, D2), lambda i: (i, 0))
    p0s = pl.BlockSpec((1, BN, D2), lambda i: (0, i, 0))
    p1s = pl.BlockSpec((1, BN, D2), lambda i: (1, i, 0))
    return pl.pallas_call(
        _stage3_body,
        grid=(NB,),
        in_specs=[
            p0s, p1s, blk(), vec(), mat(), vec(), vec(), vec(),
            pl.BlockSpec((1, 1, BN), lambda i: (i, 0, 0)),
            pl.BlockSpec((1, 1, BN), lambda i: (i + H // BN, 0, 0)),
            pl.BlockSpec((DIM, DIM), lambda i: (0, 0)),
            pl.BlockSpec((1, DIM), lambda i: (0, 0)),
            pl.BlockSpec((DIM, DOUT), lambda i: (0, 0)),
            pl.BlockSpec((1, DOUT), lambda i: (0, 0)),
        ],
        out_specs=pl.BlockSpec((G, DOUT), lambda i: (0, 0)),
        out_shape=jax.ShapeDtypeStruct((G, DOUT), jnp.float32),
        scratch_shapes=[
            pltpu.VMEM((G, DIM), jnp.float32),
            pltpu.VMEM((G, DIM), jnp.float32),
        ],
    )(pp, pp, z, b2a2, W2b_bd, b2b2, sc22, sh22, batch3d, batch3d,
      Wf1, bf1, Wf2, bf2)


def _bd(W):
    Z = jnp.zeros((DIM, DIM), jnp.float32)
    return jnp.concatenate([
        jnp.concatenate([W, Z], axis=1),
        jnp.concatenate([Z, W], axis=1),
    ], axis=0)


def _t2(v):
    return jnp.tile(v.reshape(1, DIM), (1, 2))


def kernel(x, edge_index, batch, W1a, b1a, W1b, b1b, g1, be1, rm1, rv1,
           W2a, b2a, W2b, b2b, g2, be2, rm2, rv2, Wf1, bf1, Wf2, bf2):
    eip = jnp.where(edge_index < H, 2 * edge_index, 2 * (edge_index - H) + 1)
    ei4d = eip.reshape(2, NW, KC, C)
    zeros = jnp.zeros((N_PAD, DIM), jnp.float32)
    batch3d = batch.reshape(N // BN, 1, BN)

    sc1 = g1 / jnp.sqrt(rv1 + 1e-5)
    sh1 = be1 - rm1 * sc1
    sc2 = g2 / jnp.sqrt(rv2 + 1e-5)
    sh2 = be2 - rm2 * sc2

    seg_sum = _make_seg_sum()
    y1 = _tc_in_matmul(x, W1a)
    p = seg_sum(y1.reshape(N, DIM), ei4d, zeros)
    pp = p.reshape(NC, NPH, D2)
    z = _tc_stage2(pp, y1, _t2(b1a), _bd(W1b), _t2(b1b),
                   _t2(sc1), _t2(sh1), _bd(W2a))
    p2 = seg_sum(z.reshape(N, DIM), ei4d, zeros)
    pp2 = p2.reshape(NC, NPH, D2)
    out = _tc_stage3(pp2, z, _t2(b2a), _bd(W2b), _t2(b2b),
                     _t2(sc2), _t2(sh2), batch3d,
                     Wf1, bf1.reshape(1, DIM), Wf2, bf2.reshape(1, DOUT))
    return out


# speedup vs baseline: 18.7302x; 1.0120x over previous
import functools

import jax
import jax.numpy as jnp
from jax import lax
from jax.experimental import pallas as pl
from jax.experimental.pallas import tpu as pltpu
from jax.experimental.pallas import tpu_sc as plsc

N = 10000
E = 320000
DIN = 128
DIM = 64
DOUT = 10
G = 64

NC = 2
NS = 16
NW = NC * NS
PW = E // NW
C = 200
KC = PW // C
NBUF = 5
N_PAD = 10240
ROWS_PER_TILE = N_PAD // NS


@functools.lru_cache(maxsize=1)
def _make_seg_sum():
    mesh = plsc.VectorSubcoreMesh(core_axis_name="c", subcore_axis_name="s",
                                  num_cores=NC, num_subcores=NS)

    @functools.partial(
        pl.kernel,
        mesh=mesh,
        out_type=jax.ShapeDtypeStruct((NC, N_PAD, DIM), jnp.float32),
        scratch_types=[
            pltpu.VMEM((KC, C), jnp.int32),
            pltpu.VMEM((KC, C), jnp.int32),
        ]
        + [pltpu.VMEM((C, DIM), jnp.float32) for _ in range(NBUF)]
        + [pltpu.VMEM_SHARED((N_PAD, DIM), jnp.float32)]
        + [pltpu.SemaphoreType.DMA for _ in range(2 * NBUF)],
        compiler_params=pltpu.CompilerParams(use_tc_tiling_on_sc=False),
    )
    def seg_kernel(y_hbm, ei_hbm, zero_hbm, out_hbm, src_v, dst_v, *rest):
        rows = rest[:NBUF]
        acc = rest[NBUF]
        gsem = rest[NBUF + 1:NBUF + 1 + NBUF]
        ssem = rest[NBUF + 1 + NBUF:]
        c = lax.axis_index("c")
        s = lax.axis_index("s")
        w = s * NC + c
        r0 = s * ROWS_PER_TILE
        pltpu.sync_copy(zero_hbm.at[pl.ds(r0, ROWS_PER_TILE)],
                        acc.at[pl.ds(r0, ROWS_PER_TILE)])
        pltpu.sync_copy(ei_hbm.at[0, w], src_v)
        pltpu.sync_copy(ei_hbm.at[1, w], dst_v)
        plsc.subcore_barrier()

        def gather(j, b):
            pltpu.async_copy(y_hbm.at[src_v.at[j]], rows[b], gsem[b])

        def wait_gather(j, b):
            pltpu.make_async_copy(y_hbm.at[src_v.at[j]], rows[b],
                                  gsem[b]).wait()

        def scatter(j, b):
            pltpu.async_copy(rows[b], acc.at[dst_v.at[j]], ssem[b], add=True)

        def wait_scatter(j, b):
            pltpu.make_async_copy(rows[b], acc.at[dst_v.at[j]],
                                  ssem[b]).wait()

        gather(0, 0)
        gather(1, 1)

        def body(i, carry):
            j0 = NBUF * i
            for b in range(NBUF):
                j = j0 + b
                wait_gather(j, b)
                scatter(j, b)
                jp = j - (NBUF - 2)
                bp = (b + 2) % NBUF
                bn = (b + 2) % NBUF

                @pl.when(jp >= 0)
                def _():
                    wait_scatter(jp, bp)

                @pl.when(j + 2 < KC)
                def _():
                    gather(j + 2, bn)
            return carry

        lax.fori_loop(0, KC // NBUF, body, 0)
        for j in range(KC - (NBUF - 2), KC):
            wait_scatter(j, j % NBUF)
        plsc.subcore_barrier()
        pltpu.sync_copy(acc.at[pl.ds(r0, ROWS_PER_TILE)],
                        out_hbm.at[c, pl.ds(r0, ROWS_PER_TILE)])

    return seg_kernel


H = N // 2
D2 = 2 * DIM
BN = 1000
NB = H // BN
NPH = N_PAD // 2


def _mm_body(xlo_ref, xhi_ref, w_ref, o_ref):
    ylo = jnp.dot(xlo_ref[:], w_ref[:], preferred_element_type=jnp.float32)
    yhi = jnp.dot(xhi_ref[:], w_ref[:], preferred_element_type=jnp.float32)
    o_ref[:] = jnp.concatenate([ylo, yhi], axis=-1)


def _tc_in_matmul(x, W1a):
    return pl.pallas_call(
        _mm_body,
        grid=(NB,),
        in_specs=[
            pl.BlockSpec((BN, DIN), lambda i: (i, 0)),
            pl.BlockSpec((BN, DIN), lambda i: (i + H // BN, 0)),
            pl.BlockSpec((DIN, DIM), lambda i: (0, 0)),
        ],
        out_specs=pl.BlockSpec((BN, D2), lambda i: (i, 0)),
        out_shape=jax.ShapeDtypeStruct((H, D2), jnp.float32),
    )(x, x, W1a)


def _stage2_body(p0_ref, p1_ref, y_ref, b1a_ref, w1b_ref, b1b_ref,
                 sc1_ref, sh1_ref, w2a_ref, z_ref):
    pre = p0_ref[0] + p1_ref[0] + y_ref[:] + b1a_ref[:]
    a = jnp.maximum(pre, 0.0)
    t = jnp.dot(a, w1b_ref[:], preferred_element_type=jnp.float32) + b1b_ref[:]
    t = jnp.maximum(t, 0.0)
    h = t * sc1_ref[:] + sh1_ref[:]
    z_ref[:] = jnp.dot(h, w2a_ref[:], preferred_element_type=jnp.float32)


def _tc_stage2(pp, y1, b1a2, W1b_bd, b1b2, sc12, sh12, W2a_bd):
    vec = lambda: pl.BlockSpec((1, D2), lambda i: (0, 0))
    mat = lambda: pl.BlockSpec((D2, D2), lambda i: (0, 0))
    blk = lambda: pl.BlockSpec((BN, D2), lambda i: (i, 0))
    p0s = pl.BlockSpec((1, BN, D2), lambda i: (0, i, 0))
    p1s = pl.BlockSpec((1, BN, D2), lambda i: (1, i, 0))
    return pl.pallas_call(
        _stage2_body,
        grid=(NB,),
        in_specs=[p0s, p1s, blk(), vec(), mat(), vec(), vec(), vec(), mat()],
        out_specs=blk(),
        out_shape=jax.ShapeDtypeStruct((H, D2), jnp.float32),
    )(pp, pp, y1, b1a2, W1b_bd, b1b2, sc12, sh12, W2a_bd)


def _stage3_body(p0_ref, p1_ref, z_ref, b2a_ref, w2b_ref, b2b_ref,
                 sc2_ref, sh2_ref, blo_ref, bhi_ref, wf1_ref, bf1_ref,
                 wf2_ref, bf2_ref, out_ref, sums_ref, cnt_ref):
    i = pl.program_id(0)

    @pl.when(i == 0)
    def _():
        sums_ref[:] = jnp.zeros_like(sums_ref)
        cnt_ref[:] = jnp.zeros_like(cnt_ref)

    pre = p0_ref[0] + p1_ref[0] + z_ref[:] + b2a_ref[:]
    a = jnp.maximum(pre, 0.0)
    t = jnp.dot(a, w2b_ref[:], preferred_element_type=jnp.float32) + b2b_ref[:]
    t = jnp.maximum(t, 0.0)
    h2 = t * sc2_ref[:] + sh2_ref[:]
    h2lo = h2[:, :DIM]
    h2hi = h2[:, DIM:]

    iota = lax.broadcasted_iota(jnp.int32, (BN, G), 1)
    ohlo = (blo_ref[0, 0, :][:, None] == iota).astype(jnp.float32)
    ohhi = (bhi_ref[0, 0, :][:, None] == iota).astype(jnp.float32)
    cdim = (((0,), (0,)), ((), ()))
    sums_ref[:] += (
        lax.dot_general(ohlo, h2lo, cdim, preferred_element_type=jnp.float32)
        + lax.dot_general(ohhi, h2hi, cdim,
                          preferred_element_type=jnp.float32))
    ones = jnp.ones((BN, DIM), jnp.float32)
    cnt_ref[:] += (
        lax.dot_general(ohlo, ones, cdim, preferred_element_type=jnp.float32)
        + lax.dot_general(ohhi, ones, cdim,
                          preferred_element_type=jnp.float32))

    @pl.when(i == NB - 1)
    def _():
        pooled = sums_ref[:] / jnp.maximum(cnt_ref[:], 1.0)
        h3 = jnp.maximum(
            jnp.dot(pooled, wf1_ref[:], preferred_element_type=jnp.float32)
            + bf1_ref[:], 0.0)
        out_ref[:] = (jnp.dot(h3, wf2_ref[:],
                              preferred_element_type=jnp.float32)
                      + bf2_ref[:])


def _tc_stage3(pp, z, b2a2, W2b_bd, b2b2, sc22, sh22, batch3d,
               Wf1, bf1, Wf2, bf2):
    vec = lambda: pl.BlockSpec((1, D2), lambda i: (0, 0))
    mat = lambda: pl.BlockSpec((D2, D2), lambda i: (0, 0))
    blk = lambda: pl.BlockSpec((BN, D2), lambda i: (i, 0))
    p0s = pl.BlockSpec((1, BN, D2), lambda i: (0, i, 0))
    p1s = pl.BlockSpec((1, BN, D2), lambda i: (1, i, 0))
    return pl.pallas_call(
        _stage3_body,
        grid=(NB,),
        in_specs=[
            p0s, p1s, blk(), vec(), mat(), vec(), vec(), vec(),
            pl.BlockSpec((1, 1, BN), lambda i: (i, 0, 0)),
            pl.BlockSpec((1, 1, BN), lambda i: (i + H // BN, 0, 0)),
            pl.BlockSpec((DIM, DIM), lambda i: (0, 0)),
            pl.BlockSpec((1, DIM), lambda i: (0, 0)),
            pl.BlockSpec((DIM, DOUT), lambda i: (0, 0)),
            pl.BlockSpec((1, DOUT), lambda i: (0, 0)),
        ],
        out_specs=pl.BlockSpec((G, DOUT), lambda i: (0, 0)),
        out_shape=jax.ShapeDtypeStruct((G, DOUT), jnp.float32),
        scratch_shapes=[
            pltpu.VMEM((G, DIM), jnp.float32),
            pltpu.VMEM((G, DIM), jnp.float32),
        ],
    )(pp, pp, z, b2a2, W2b_bd, b2b2, sc22, sh22, batch3d, batch3d,
      Wf1, bf1, Wf2, bf2)


def _bd(W):
    Z = jnp.zeros((DIM, DIM), jnp.float32)
    return jnp.concatenate([
        jnp.concatenate([W, Z], axis=1),
        jnp.concatenate([Z, W], axis=1),
    ], axis=0)


def _t2(v):
    return jnp.tile(v.reshape(1, DIM), (1, 2))


def kernel(x, edge_index, batch, W1a, b1a, W1b, b1b, g1, be1, rm1, rv1,
           W2a, b2a, W2b, b2b, g2, be2, rm2, rv2, Wf1, bf1, Wf2, bf2):
    eip = jnp.where(edge_index < H, 2 * edge_index, 2 * (edge_index - H) + 1)
    ei4d = eip.reshape(2, NW, KC, C)
    zeros = jnp.zeros((N_PAD, DIM), jnp.float32)
    batch3d = batch.reshape(N // BN, 1, BN)

    sc1 = g1 / jnp.sqrt(rv1 + 1e-5)
    sh1 = be1 - rm1 * sc1
    sc2 = g2 / jnp.sqrt(rv2 + 1e-5)
    sh2 = be2 - rm2 * sc2

    seg_sum = _make_seg_sum()
    y1 = _tc_in_matmul(x, W1a)
    p = seg_sum(y1.reshape(N, DIM), ei4d, zeros)
    pp = p.reshape(NC, NPH, D2)
    z = _tc_stage2(pp, y1, _t2(b1a), _bd(W1b), _t2(b1b),
                   _t2(sc1), _t2(sh1), _bd(W2a))
    p2 = seg_sum(z.reshape(N, DIM), ei4d, zeros)
    pp2 = p2.reshape(NC, NPH, D2)
    out = _tc_stage3(pp2, z, _t2(b2a), _bd(W2b), _t2(b2b),
                     _t2(sc2), _t2(sh2), batch3d,
                     Wf1, bf1.reshape(1, DIM), Wf2, bf2.reshape(1, DOUT))
    return out
